# Initial kernel scaffold; baseline (speedup 1.0000x reference)
import jax
import jax.numpy as jnp
from jax.experimental import pallas as pl


def kernel(x_d, di_gua, di_cos, di_sem, W_t1, b_t1, W_t2, b_t2, W_s1, b_s1, W_s2, b_s2, W_g1, b_g1, W_g2, b_g2, fc1_W, fc1_b, fc2_W, fc2_b, cnn_W, cnn_b, di_gua_edges, di_cos_edges, di_sem_edges):
    raise NotImplementedError("write your pallas kernel here")


# speedup vs baseline: 39.1864x; 39.1864x over previous
import functools

import jax
import jax.numpy as jnp
from jax import lax
from jax.experimental import pallas as pl
from jax.experimental.pallas import tpu as pltpu
from jax.experimental.pallas import tpu_sc as plsc

N = 884
FD = 128
E = 56576
NN = N * N
NNP = 781568
NS = 16
L = 16
EPT = E // NS
ZCH = NNP // NS
TAIL = NN - 15 * ZCH


def _sc_body(src0, dst0, src1, dst1, src2, dst2, out,
             src_v, dst_v, idx_v, ones_v, stage_v, acc):
    c = lax.axis_index("c")
    s = lax.axis_index("s")

    def fill_zeros():
        def fill(i, _):
            stage_v[pl.ds(i * L, L)] = jnp.zeros((L,), jnp.float32)
            return 0
        lax.fori_loop(0, ZCH // L, fill, 0)

    def fill1(i, _):
        ones_v[pl.ds(i * L, L)] = jnp.ones((L,), jnp.float32)
        return 0
    lax.fori_loop(0, EPT // L, fill1, 0)
    fill_zeros()

    def zero_acc():
        pltpu.sync_copy(stage_v, acc.at[pl.ds(s * ZCH, ZCH)])

    def histogram(src_h, dst_h):
        base = s * EPT
        pltpu.sync_copy(src_h.at[pl.ds(base, EPT)], src_v)
        pltpu.sync_copy(dst_h.at[pl.ds(base, EPT)], dst_v)

        def body(i, _):
            sl = pl.ds(i * L, L)
            idx_v[sl] = src_v[sl] * N + dst_v[sl]
            return 0
        lax.fori_loop(0, EPT // L, body, 0)
        pltpu.sync_copy(ones_v, acc.at[idx_v], add=True)

    def copy_out(vbase):
        pltpu.sync_copy(acc.at[pl.ds(s * ZCH, ZCH)], stage_v)

        @pl.when(s < 15)
        def _():
            pltpu.sync_copy(stage_v, out.at[pl.ds(vbase + s * ZCH, ZCH)])

        @pl.when(s == 15)
        def _():
            pltpu.sync_copy(stage_v.at[pl.ds(0, TAIL)],
                            out.at[pl.ds(vbase + 15 * ZCH, TAIL)])

    zero_acc()
    plsc.subcore_barrier()

    @pl.when(c == 0)
    def _():
        histogram(src0, dst0)

    @pl.when(c == 1)
    def _():
        histogram(src2, dst2)

    plsc.subcore_barrier()

    @pl.when(c == 0)
    def _():
        copy_out(0)

    @pl.when(c == 1)
    def _():
        copy_out(2 * NN)

    plsc.subcore_barrier()

    @pl.when(c == 0)
    def _():
        fill_zeros()
        zero_acc()

    plsc.subcore_barrier()

    @pl.when(c == 0)
    def _():
        histogram(src1, dst1)

    plsc.subcore_barrier()

    @pl.when(c == 0)
    def _():
        copy_out(NN)


@functools.cache
def _sc_histogram():
    return pl.kernel(
        _sc_body,
        mesh=plsc.VectorSubcoreMesh(core_axis_name="c", subcore_axis_name="s"),
        out_type=jax.ShapeDtypeStruct((3 * NN,), jnp.float32),
        scratch_types=[
            pltpu.VMEM((EPT,), jnp.int32),
            pltpu.VMEM((EPT,), jnp.int32),
            pltpu.VMEM((EPT,), jnp.int32),
            pltpu.VMEM((EPT,), jnp.float32),
            pltpu.VMEM((ZCH,), jnp.float32),
            pltpu.VMEM_SHARED((NNP,), jnp.float32),
        ],
    )


def _tc_body(C_ref, dg_ref, dc_ref, dsm_ref, x_ref, W1_ref, b1_ref,
             W2_ref, b2_ref, fc1W_ref, fc1b_ref, fc2W_ref, fc2b_ref,
             cnnW_ref, cnnb_ref, out_ref):
    Xt = x_ref[...].T
    di_refs = (dg_ref, dc_ref, dsm_ref)
    Zs = []
    for v in range(3):
        B = C_ref[v] * di_refs[v][...]
        deg = jnp.sum(B, axis=0, keepdims=True) + 1.0
        dinv = lax.rsqrt(deg)
        W1 = W1_ref[v]
        G = jnp.dot(W1.T, Xt, preferred_element_type=jnp.float32) * dinv
        Z1 = jnp.maximum(
            dinv * (jnp.dot(G, B, preferred_element_type=jnp.float32) + G)
            + b1_ref[v], 0.0)
        W2 = W2_ref[v]
        G2 = jnp.dot(W2.T, Z1, preferred_element_type=jnp.float32) * dinv
        Z2 = jnp.maximum(
            dinv * (jnp.dot(G2, B, preferred_element_type=jnp.float32) + G2)
            + b2_ref[v], 0.0)
        Zs += [Z1, Z2]

    inv = 1.0 / (N * FD)
    fc1W = fc1W_ref[...]
    h1 = fc1b_ref[...]
    for cc in range(6):
        h1 = h1 + (jnp.sum(Zs[cc]) * inv) * fc1W[cc:cc + 1, :]
    h1 = jnp.maximum(h1, 0.0)
    h2 = jnp.dot(h1, fc2W_ref[...],
                 preferred_element_type=jnp.float32) + fc2b_ref[...]
    att = 1.0 / (1.0 + jnp.exp(-h2))
    coef = att * cnnW_ref[...]

    acc = coef[0, 0] * Zs[0]
    for cc in range(1, 6):
        acc = acc + coef[0, cc] * Zs[cc]
    out_ref[...] = acc.T + cnnb_ref[0, 0]


def kernel(x_d, di_gua, di_cos, di_sem, W_t1, b_t1, W_t2, b_t2, W_s1, b_s1,
           W_s2, b_s2, W_g1, b_g1, W_g2, b_g2, fc1_W, fc1_b, fc2_W, fc2_b,
           cnn_W, cnn_b, di_gua_edges, di_cos_edges, di_sem_edges):
    counts = _sc_histogram()(
        di_gua_edges[0], di_gua_edges[1],
        di_cos_edges[0], di_cos_edges[1],
        di_sem_edges[0], di_sem_edges[1],
    )
    C3 = counts.reshape(3, N, N)
    W1s = jnp.stack([W_t1, W_s1, W_g1])
    W2s = jnp.stack([W_t2, W_s2, W_g2])
    b1s = jnp.stack([b_t1, b_s1, b_g1])[:, :, None]
    b2s = jnp.stack([b_t2, b_s2, b_g2])[:, :, None]
    out = pl.pallas_call(
        _tc_body,
        out_shape=jax.ShapeDtypeStruct((N, FD), jnp.float32),
    )(C3, di_gua, di_cos, di_sem, x_d, W1s, b1s, W2s, b2s,
      fc1_W, fc1_b.reshape(1, -1), fc2_W, fc2_b.reshape(1, -1),
      cnn_W.reshape(1, -1), cnn_b.reshape(1, 1))
    return out


# speedup vs baseline: 78.9303x; 2.0142x over previous
import functools

import jax
import jax.numpy as jnp
from jax import lax
from jax.experimental import pallas as pl
from jax.experimental.pallas import tpu as pltpu
from jax.experimental.pallas import tpu_sc as plsc

N = 884
FD = 128
E = 56576
K = 7
PR = 888
PW = PR * 128
NNF = K * PW
SLAB = K * PR
NS = 16
L = 16
EPT = E // NS
ZCH = NNF // NS
NIT = EPT // L


def _sc_body(e0, e1, e2, out, src_v, dst_v, idx_v, ones_v, stage_v, acc, sem,
             fsem):
    c = lax.axis_index("c")
    s = lax.axis_index("s")
    zero16 = jnp.zeros((L,), jnp.float32)
    one16 = jnp.ones((L,), jnp.float32)
    nz = ZCH // L

    def fillz(i, _):
        for j in range(8):
            stage_v[pl.ds((i * 8 + j) * L, L)] = zero16
        return 0
    lax.fori_loop(0, nz // 8, fillz, 0)
    for j in range((nz // 8) * 8, nz):
        stage_v[pl.ds(j * L, L)] = zero16

    def fillo(i, _):
        for j in range(8):
            ones_v[pl.ds((i * 8 + j) * L, L)] = one16
        return 0
    lax.fori_loop(0, NIT // 8, fillo, 0)
    for j in range((NIT // 8) * 8, NIT):
        ones_v[pl.ds(j * L, L)] = one16

    pltpu.sync_copy(stage_v, acc.at[pl.ds(s * ZCH, ZCH)])
    plsc.subcore_barrier()

    def histogram(e):
        base = s * EPT
        cp1 = pltpu.make_async_copy(e.at[pl.ds(base, EPT)], src_v, sem)
        cp2 = pltpu.make_async_copy(e.at[pl.ds(E + base, EPT)], dst_v, sem)
        cp1.start()
        cp2.start()
        cp1.wait()
        cp2.wait()

        def idx16(sl):
            d = dst_v[sl]
            idx_v[sl] = ((d >> 7) * PW + (src_v[sl] << 7)) + (d & 127)

        def body(i, _):
            for j in range(4):
                idx16(pl.ds((i * 4 + j) * L, L))
            return 0
        lax.fori_loop(0, NIT // 4, body, 0)
        for j in range((NIT // 4) * 4, NIT):
            idx16(pl.ds(j * L, L))
        pltpu.sync_copy(ones_v, acc.at[idx_v], add=True)

    @pl.when(c == 0)
    def _():
        histogram(e0)

    @pl.when(c == 1)
    def _():
        histogram(e2)

    plsc.subcore_barrier()

    pltpu.sync_copy(acc.at[pl.ds(s * ZCH, ZCH)], stage_v)
    plsc.subcore_barrier()

    vbase = c * (2 * NNF)
    flush = pltpu.make_async_copy(
        stage_v, out.at[pl.ds(vbase + s * ZCH, ZCH)], fsem)
    flush.start()

    @pl.when(c == 0)
    def _():
        histogram(e1)

    flush.wait()
    plsc.subcore_barrier()

    @pl.when(c == 0)
    def _():
        pltpu.sync_copy(acc.at[pl.ds(s * ZCH, ZCH)], stage_v)
        pltpu.sync_copy(stage_v, out.at[pl.ds(NNF + s * ZCH, ZCH)])


@functools.cache
def _sc_histogram():
    return pl.kernel(
        _sc_body,
        mesh=plsc.VectorSubcoreMesh(core_axis_name="c", subcore_axis_name="s"),
        out_type=jax.ShapeDtypeStruct((3 * NNF,), jnp.float32),
        scratch_types=[
            pltpu.VMEM((EPT,), jnp.int32),
            pltpu.VMEM((EPT,), jnp.int32),
            pltpu.VMEM((EPT,), jnp.int32),
            pltpu.VMEM((EPT,), jnp.float32),
            pltpu.VMEM((ZCH,), jnp.float32),
            pltpu.VMEM_SHARED((NNF,), jnp.float32),
            pltpu.SemaphoreType.DMA,
            pltpu.SemaphoreType.DMA,
        ],
    )


def _tc_body(cnt_hbm, dg, dc, dsm, x_ref,
             Wt1, Wt2, Ws1, Ws2, Wg1, Wg2,
             bt1, bt2, bs1, bs2, bg1, bg2,
             fc1W_ref, fc1b_ref, fc2W_ref, fc2b_ref, cnnW_ref, cnnb_ref,
             out_ref, cb0, cb1, cb2, db0, db1, db2, sem):
    cbufs = (cb0, cb1, cb2)
    dbufs = (db0, db1, db2)
    di_h = (dg, dc, dsm)
    descs = []
    for v in range(3):
        dc_ = pltpu.make_async_copy(
            cnt_hbm.at[pl.ds(v * SLAB, SLAB), :], cbufs[v], sem.at[v])
        dd_ = pltpu.make_async_copy(di_h[v], dbufs[v], sem.at[3 + v])
        dc_.start()
        dd_.start()
        descs.append((dc_, dd_))

    W1s = (Wt1, Ws1, Wg1)
    W2s = (Wt2, Ws2, Wg2)
    b1s = (bt1, bs1, bg1)
    b2s = (bt2, bs2, bg2)
    Xt = x_ref[...].T
    Zs = []
    P0 = None
    for v in range(3):
        descs[v][0].wait()
        descs[v][1].wait()
        Praw = [cbufs[v][pl.ds(k * PR, N), :] for k in range(K)]
        if v == 0:
            P0 = Praw
        P = [Praw[k] - P0[k] for k in range(K)] if v == 1 else Praw
        D = dbufs[v][...]
        Dp = jnp.concatenate(
            [D, jnp.zeros((N, K * 128 - N), jnp.float32)], axis=1)
        Bk = [P[k] * Dp[:, k * 128:(k + 1) * 128] for k in range(K)]
        deg = jnp.concatenate(
            [jnp.sum(Bk[k], axis=0, keepdims=True) for k in range(K)],
            axis=1)[:, :N] + 1.0
        dinv = lax.rsqrt(deg)
        G = jnp.dot(W1s[v][...].T, Xt,
                    preferred_element_type=jnp.float32) * dinv
        GB = jnp.concatenate(
            [jnp.dot(G, Bk[k], preferred_element_type=jnp.float32)
             for k in range(K)], axis=1)[:, :N]
        Z1 = jnp.maximum(dinv * (GB + G) + b1s[v][...], 0.0)
        G2 = jnp.dot(W2s[v][...].T, Z1,
                     preferred_element_type=jnp.float32) * dinv
        GB2 = jnp.concatenate(
            [jnp.dot(G2, Bk[k], preferred_element_type=jnp.float32)
             for k in range(K)], axis=1)[:, :N]
        Z2 = jnp.maximum(dinv * (GB2 + G2) + b2s[v][...], 0.0)
        Zs += [Z1, Z2]

    inv = 1.0 / (N * FD)
    fc1W = fc1W_ref[...]
    h1 = fc1b_ref[...]
    for cc in range(6):
        h1 = h1 + (jnp.sum(Zs[cc]) * inv) * fc1W[cc:cc + 1, :]
    h1 = jnp.maximum(h1, 0.0)
    h2 = jnp.dot(h1, fc2W_ref[...],
                 preferred_element_type=jnp.float32) + fc2b_ref[...]
    att = 1.0 / (1.0 + jnp.exp(-h2))
    coef = att * cnnW_ref[...]

    acc = coef[0, 0] * Zs[0]
    for cc in range(1, 6):
        acc = acc + coef[0, cc] * Zs[cc]
    out_ref[...] = acc.T + cnnb_ref[0, 0]


def kernel(x_d, di_gua, di_cos, di_sem, W_t1, b_t1, W_t2, b_t2, W_s1, b_s1,
           W_s2, b_s2, W_g1, b_g1, W_g2, b_g2, fc1_W, fc1_b, fc2_W, fc2_b,
           cnn_W, cnn_b, di_gua_edges, di_cos_edges, di_sem_edges):
    counts = _sc_histogram()(di_gua_edges.reshape(-1), di_cos_edges.reshape(-1),
                             di_sem_edges.reshape(-1))
    counts = counts.reshape(3 * SLAB, 128)
    anyspec = pl.BlockSpec(memory_space=pl.ANY)
    vspec = pl.BlockSpec(memory_space=pltpu.MemorySpace.VMEM)
    out = pl.pallas_call(
        _tc_body,
        out_shape=jax.ShapeDtypeStruct((N, FD), jnp.float32),
        in_specs=[anyspec] * 4 + [vspec] * 19,
        out_specs=vspec,
        scratch_shapes=(
            [pltpu.VMEM((SLAB, 128), jnp.float32)] * 3
            + [pltpu.VMEM((N, N), jnp.float32)] * 3
            + [pltpu.SemaphoreType.DMA((6,))]
        ),
    )(counts, di_gua, di_cos, di_sem, x_d,
      W_t1, W_t2, W_s1, W_s2, W_g1, W_g2,
      b_t1.reshape(FD, 1), b_t2.reshape(FD, 1), b_s1.reshape(FD, 1),
      b_s2.reshape(FD, 1), b_g1.reshape(FD, 1), b_g2.reshape(FD, 1),
      fc1_W, fc1_b.reshape(1, -1), fc2_W, fc2_b.reshape(1, -1),
      cnn_W.reshape(1, -1), cnn_b.reshape(1, 1))
    return out
